# baseline (speedup 1.0000x reference)
import functools

import jax
import jax.numpy as jnp
from jax import lax
from jax.experimental import pallas as pl
from jax.experimental.pallas import tpu as pltpu
from jax.experimental.pallas import tpu_sc as plsc

N = 10000
E = 320000
DN = 128
DE = 16
DG = 128
H1 = 256
H2 = 128

TE = 4000
GE = E // TE
TN = 2000
GN = N // TN
CHUNK = 128
NCHUNKS = E // CHUNK
NSUB = 16
CPS_BASE = NCHUNKS // NSUB
CPS_REM = NCHUNKS % NSUB

_DEBUG_JNP_SEG = False
_DEBUG_JNP_EDGE = False


def _zeros32():
    return jnp.zeros((N // NSUB, H2), jnp.float32)


def _edge_body(edges_ref, we1_ref, be1_ref, we2_ref, be2_ref,
               e1ab_ref, e2_ref, s1_ref, s2_ref):
    i = pl.program_id(0)
    e1 = jnp.maximum(
        jnp.dot(edges_ref[...], we1_ref[...],
                preferred_element_type=jnp.float32) + be1_ref[...], 0.0)
    e1ab_ref[0] = e1[:, :H1 // 2]
    e1ab_ref[1] = e1[:, H1 // 2:]
    e2 = jnp.maximum(
        jnp.dot(e1, we2_ref[...],
                preferred_element_type=jnp.float32) + be2_ref[...], 0.0)
    e2_ref[...] = e2

    @pl.when(i == 0)
    def _():
        s1_ref[...] = jnp.zeros_like(s1_ref)
        s2_ref[...] = jnp.zeros_like(s2_ref)

    s1_ref[...] += jnp.sum(e1, axis=0, keepdims=True)
    s2_ref[...] += jnp.sum(e2, axis=0, keepdims=True)


def _edge_stage(edges, W_e1, b_e1, W_e2, b_e2):
    return pl.pallas_call(
        _edge_body,
        grid=(GE,),
        in_specs=[
            pl.BlockSpec((TE, DE), lambda i: (i, 0)),
            pl.BlockSpec((DE, H1), lambda i: (0, 0)),
            pl.BlockSpec((1, H1), lambda i: (0, 0)),
            pl.BlockSpec((H1, H2), lambda i: (0, 0)),
            pl.BlockSpec((1, H2), lambda i: (0, 0)),
        ],
        out_specs=[
            pl.BlockSpec((2, TE, H1 // 2), lambda i: (0, i, 0)),
            pl.BlockSpec((TE, H2), lambda i: (i, 0)),
            pl.BlockSpec((1, H1), lambda i: (0, 0)),
            pl.BlockSpec((1, H2), lambda i: (0, 0)),
        ],
        out_shape=[
            jax.ShapeDtypeStruct((2, E, H1 // 2), jnp.float32),
            jax.ShapeDtypeStruct((E, H2), jnp.float32),
            jax.ShapeDtypeStruct((1, H1), jnp.float32),
            jax.ShapeDtypeStruct((1, H2), jnp.float32),
        ],
        compiler_params=pltpu.CompilerParams(
            dimension_semantics=("arbitrary",)),
    )(edges, W_e1, b_e1.reshape(1, H1), W_e2, b_e2.reshape(1, H2))


def _zero_fill(ref, rows, width):
    z = jnp.zeros((16,), jnp.float32)

    def body(i, _):
        for k in range(width // 16):
            ref[i, pl.ds(k * 16, 16)] = z
        return 0

    lax.fori_loop(0, rows, body, 0, unroll=False)


NBUF = 2
STEPS = (CPS_BASE + (1 if CPS_REM else 0) + NBUF - 1) // NBUF


def _pipelined_scatter(idx_hbm, data_slice, idx_bufs, data_bufs,
                       isems, dsems, ssems, acc, s):
    start = s * CPS_BASE + jnp.minimum(s, CPS_REM)
    cnt = CPS_BASE + jnp.where(s < CPS_REM, 1, 0)
    end = start + cnt

    def issue(b, k):
        pltpu.async_copy(idx_hbm.at[pl.ds(k * CHUNK, CHUNK)],
                         idx_bufs[b], isems[b])
        if data_bufs is not None:
            pltpu.async_copy(data_slice(k), data_bufs[b], dsems[b])

    def wait(b):
        pltpu.make_async_copy(idx_hbm.at[pl.ds(0, CHUNK)],
                              idx_bufs[b], isems[b]).wait()
        if data_bufs is not None:
            pltpu.make_async_copy(data_slice(start), data_bufs[b],
                                  dsems[b]).wait()

    def _src(b):
        return data_bufs[b] if data_bufs is not None else data_slice

    def scatter(b):
        pltpu.async_copy(_src(b), acc.at[idx_bufs[b]], ssems[b], add=True)

    def wait_scatter(b):
        pltpu.make_async_copy(_src(b), acc.at[idx_bufs[b]], ssems[b]).wait()

    for b in range(NBUF):
        @pl.when(start + b < end)
        def _(b=b):
            issue(b, start + b)

    def step_body(q, _):
        base = start + q * NBUF
        for b in range(NBUF):
            k = base + b

            @pl.when(k < end)
            def _(b=b, k=k):
                wait(b)
                scatter(b)

        for b in range(NBUF):
            k = base + b

            @pl.when(k + NBUF < end)
            def _(b=b, k=k):
                wait_scatter(b)
                issue(b, k + NBUF)

        return 0

    lax.fori_loop(0, STEPS, step_body, 0, unroll=False)

    for b in range(NBUF):
        @pl.when(start + b < end)
        def _(b=b):
            wait_scatter(b)


def _seg_sum_e1(data, idx2d, which):
    mesh = plsc.VectorSubcoreMesh(core_axis_name="c", subcore_axis_name="s")
    D2 = H1 // 2

    @functools.partial(
        pl.kernel,
        mesh=mesh,
        out_type=jax.ShapeDtypeStruct((2, N, D2), jnp.float32),
        scratch_types=[
            [pltpu.VMEM((CHUNK, D2), jnp.float32)] * NBUF,
            [pltpu.VMEM((CHUNK,), jnp.int32)] * NBUF,
            pltpu.VMEM_SHARED((N, D2), jnp.float32),
            [pltpu.SemaphoreType.DMA] * NBUF,
            [pltpu.SemaphoreType.DMA] * NBUF,
            [pltpu.SemaphoreType.DMA] * NBUF,
        ],
        name=f"seg_sum_e1_{which}",
    )
    def k(data_hbm, idx_hbm, z_hbm, out_hbm, data_vs, idx_vs, acc,
          iss, dss, sss):
        c = lax.axis_index("c")
        s = lax.axis_index("s")
        pltpu.sync_copy(z_hbm, acc.at[pl.ds(s * (N // NSUB), N // NSUB), :])
        plsc.subcore_barrier()

        def data_slice(k):
            return data_hbm.at[c, pl.ds(k * CHUNK, CHUNK), :]

        _pipelined_scatter(idx_hbm, data_slice, idx_vs, data_vs,
                           iss, dss, sss, acc, s)
        plsc.subcore_barrier()

        @pl.when(s == 0)
        def _():
            pltpu.sync_copy(acc, out_hbm.at[c])

    return k(data, idx2d, _zeros32())


def _seg_sum_e2(data, ridx2d, sidx2d):
    mesh = plsc.VectorSubcoreMesh(core_axis_name="c", subcore_axis_name="s")

    @functools.partial(
        pl.kernel,
        mesh=mesh,
        out_type=jax.ShapeDtypeStruct((2, N, H2), jnp.float32),
        scratch_types=[
            [pltpu.VMEM((CHUNK, H2), jnp.float32)] * NBUF,
            [pltpu.VMEM((CHUNK,), jnp.int32)] * NBUF,
            pltpu.VMEM_SHARED((N, H2), jnp.float32),
            [pltpu.SemaphoreType.DMA] * NBUF,
            [pltpu.SemaphoreType.DMA] * NBUF,
            [pltpu.SemaphoreType.DMA] * NBUF,
        ],
        name="seg_sum_e2",
    )
    def k(data_hbm, ridx_hbm, sidx_hbm, z_hbm, out_hbm, data_vs, idx_vs,
          acc, iss, dss, sss):
        c = lax.axis_index("c")
        s = lax.axis_index("s")
        pltpu.sync_copy(z_hbm, acc.at[pl.ds(s * (N // NSUB), N // NSUB), :])
        plsc.subcore_barrier()

        def data_slice(k):
            return data_hbm.at[pl.ds(k * CHUNK, CHUNK), :]

        @pl.when(c == 0)
        def _():
            _pipelined_scatter(ridx_hbm, data_slice, idx_vs, data_vs,
                               iss, dss, sss, acc, s)

        @pl.when(c == 1)
        def _():
            _pipelined_scatter(sidx_hbm, data_slice, idx_vs, data_vs,
                               iss, dss, sss, acc, s)

        plsc.subcore_barrier()

        @pl.when(s == 0)
        def _():
            pltpu.sync_copy(acc, out_hbm.at[c])

    return k(data, ridx2d, sidx2d, _zeros32())


def _counts(ridx2d, sidx2d):
    mesh = plsc.VectorSubcoreMesh(core_axis_name="c", subcore_axis_name="s")

    @functools.partial(
        pl.kernel,
        mesh=mesh,
        out_type=jax.ShapeDtypeStruct((2, N, H2), jnp.float32),
        scratch_types=[
            pltpu.VMEM((CHUNK, H2), jnp.float32),
            [pltpu.VMEM((CHUNK,), jnp.int32)] * NBUF,
            [pltpu.SemaphoreType.DMA] * NBUF,
            [pltpu.SemaphoreType.DMA] * NBUF,
            pltpu.VMEM_SHARED((N, H2), jnp.float32),
        ],
        name="idx_counts",
    )
    def k(ridx_hbm, sidx_hbm, z_hbm, out_hbm, ones_v, idx_vs, iss, sss, acc):
        c = lax.axis_index("c")
        s = lax.axis_index("s")
        one = jnp.ones((16,), jnp.float32)

        def ones_body(i, _):
            for kk in range(H2 // 16):
                ones_v[i, pl.ds(kk * 16, 16)] = one
            return 0

        lax.fori_loop(0, CHUNK, ones_body, 0, unroll=False)
        pltpu.sync_copy(z_hbm, acc.at[pl.ds(s * (N // NSUB), N // NSUB), :])
        plsc.subcore_barrier()

        @pl.when(c == 0)
        def _():
            _pipelined_scatter(ridx_hbm, ones_v, idx_vs, None,
                               iss, None, sss, acc, s)

        @pl.when(c == 1)
        def _():
            _pipelined_scatter(sidx_hbm, ones_v, idx_vs, None,
                               iss, None, sss, acc, s)

        plsc.subcore_barrier()

        @pl.when(s == 0)
        def _():
            pltpu.sync_copy(acc, out_hbm.at[c])

    return k(ridx2d, sidx2d, _zeros32())


def _node_body(nodes_ref, g_ref, i1_ref, o1_ref, io2_ref,
               cnt_ref, s1_ref, s2_ref,
               wn1_ref, win1_ref, wout1_ref, bn1_ref,
               wg1_ref, wgn1_ref, wge1_ref, bg1_ref,
               wn2_ref, win2_ref, wout2_ref, bn2_ref,
               wg2_ref, wgn2_ref, wge2_ref, bg2_ref,
               n2_ref, g2_ref, sn1_ref, sn2_ref):
    i = pl.program_id(0)
    rr = 1.0 / jnp.maximum(cnt_ref[0, :, 0:1], 1.0)
    rs = 1.0 / jnp.maximum(cnt_ref[1, :, 0:1], 1.0)

    def dot(a, b):
        return jnp.dot(a, b, preferred_element_type=jnp.float32)

    h = H1 // 2
    n1 = dot(nodes_ref[...], wn1_ref[...])
    n1 += dot(i1_ref[0] * rr, win1_ref[0:h]) + dot(i1_ref[1] * rr, win1_ref[h:])
    n1 += dot(o1_ref[0] * rs, wout1_ref[0:h]) + dot(o1_ref[1] * rs, wout1_ref[h:])
    n1 = jnp.maximum(n1 + bn1_ref[...], 0.0)

    n2 = dot(n1, wn2_ref[...])
    n2 += dot(io2_ref[0] * rr, win2_ref[...])
    n2 += dot(io2_ref[1] * rs, wout2_ref[...])
    n2 = jnp.maximum(n2 + bn2_ref[...], 0.0)
    n2_ref[...] = n2

    @pl.when(i == 0)
    def _():
        sn1_ref[...] = jnp.zeros_like(sn1_ref)
        sn2_ref[...] = jnp.zeros_like(sn2_ref)

    sn1_ref[...] += jnp.sum(n1, axis=0, keepdims=True)
    sn2_ref[...] += jnp.sum(n2, axis=0, keepdims=True)

    @pl.when(i == GN - 1)
    def _():
        g1 = dot(g_ref[...], wg1_ref[...])
        g1 += dot(sn1_ref[...] * (1.0 / N), wgn1_ref[...])
        g1 += dot(s1_ref[...] * (1.0 / E), wge1_ref[...])
        g1 = jnp.maximum(g1 + bg1_ref[...], 0.0)
        g2 = dot(g1, wg2_ref[...])
        g2 += dot(sn2_ref[...] * (1.0 / N), wgn2_ref[...])
        g2 += dot(s2_ref[...] * (1.0 / E), wge2_ref[...])
        g2_ref[...] = jnp.maximum(g2 + bg2_ref[...], 0.0)


def _node_stage(nodes, globals_, i1, o1, io2, cnt, s1, s2,
                W_n1, W_in1, W_out1, b_n1, W_g1, W_gn1, W_ge1, b_g1,
                W_n2, W_in2, W_out2, b_n2, W_g2, W_gn2, W_ge2, b_g2):
    full = lambda a, b: pl.BlockSpec((a, b), lambda i: (0, 0))
    row = lambda w: pl.BlockSpec((TN, w), lambda i: (i, 0))
    split = lambda w: pl.BlockSpec((2, TN, w), lambda i: (0, i, 0))
    return pl.pallas_call(
        _node_body,
        grid=(GN,),
        in_specs=[
            row(DN), full(1, DG),
            split(H1 // 2), split(H1 // 2), split(H2),
            pl.BlockSpec((2, TN, H2), lambda i: (0, i, 0)),
            full(1, H1), full(1, H2),
            full(DN, H1), full(H1, H1), full(H1, H1), full(1, H1),
            full(DG, H1), full(H1, H1), full(H1, H1), full(1, H1),
            full(H1, H2), full(H2, H2), full(H2, H2), full(1, H2),
            full(H1, H2), full(H2, H2), full(H2, H2), full(1, H2),
        ],
        out_specs=[
            pl.BlockSpec((TN, H2), lambda i: (i, 0)),
            pl.BlockSpec((1, H2), lambda i: (0, 0)),
        ],
        out_shape=[
            jax.ShapeDtypeStruct((N, H2), jnp.float32),
            jax.ShapeDtypeStruct((1, H2), jnp.float32),
        ],
        scratch_shapes=[
            pltpu.VMEM((1, H1), jnp.float32),
            pltpu.VMEM((1, H2), jnp.float32),
        ],
        compiler_params=pltpu.CompilerParams(
            dimension_semantics=("arbitrary",)),
    )(nodes, globals_.reshape(1, DG), i1, o1, io2, cnt, s1, s2,
      W_n1, W_in1, W_out1, b_n1.reshape(1, H1),
      W_g1, W_gn1, W_ge1, b_g1.reshape(1, H1),
      W_n2, W_in2, W_out2, b_n2.reshape(1, H2),
      W_g2, W_gn2, W_ge2, b_g2.reshape(1, H2))


def kernel(nodes, edges, globals_, senders, receivers,
           W_e1, b_e1, W_n1, W_in1, W_out1, b_n1, W_g1, W_gn1, W_ge1, b_g1,
           W_e2, b_e2, W_n2, W_in2, W_out2, b_n2, W_g2, W_gn2, W_ge2, b_g2):
    ridx2d = receivers.astype(jnp.int32)
    sidx2d = senders.astype(jnp.int32)

    cnt_sc = _counts(ridx2d, sidx2d)
    if _DEBUG_JNP_EDGE:
        e1 = jax.nn.relu(edges @ W_e1 + b_e1)
        e2 = jax.nn.relu(e1 @ W_e2 + b_e2)
        e1ab = jnp.stack([e1[:, :H1 // 2], e1[:, H1 // 2:]], axis=0)
        s1 = jnp.sum(e1, axis=0, keepdims=True)
        s2 = jnp.sum(e2, axis=0, keepdims=True)
    else:
        e1ab, e2, s1, s2 = _edge_stage(edges, W_e1, b_e1, W_e2, b_e2)
    if _DEBUG_JNP_SEG:
        recv = receivers.astype(jnp.int32)
        send = senders.astype(jnp.int32)
        e1 = jnp.concatenate([e1ab[0], e1ab[1]], axis=1)
        seg = lambda d, x: jax.ops.segment_sum(d, x, num_segments=N)
        sp = lambda x: jnp.stack([x[:, :H1 // 2], x[:, H1 // 2:]], axis=0)
        i1 = sp(seg(e1, recv))
        o1 = sp(seg(e1, send))
        io2 = jnp.stack([seg(e2, recv), seg(e2, send)], axis=0)
        ones = jnp.ones((E,), jnp.float32)
        cnt = jnp.stack([jnp.tile(seg(ones, recv)[:, None], (1, H2)),
                         jnp.tile(seg(ones, send)[:, None], (1, H2))], axis=0)
    else:
        i1 = _seg_sum_e1(e1ab, ridx2d, "recv")
        o1 = _seg_sum_e1(e1ab, sidx2d, "send")
        io2 = _seg_sum_e2(e2, ridx2d, sidx2d)
        cnt = cnt_sc

    n2, g2 = _node_stage(
        nodes, globals_, i1, o1, io2, cnt, s1, s2,
        W_n1, W_in1, W_out1, b_n1, W_g1, W_gn1, W_ge1, b_g1,
        W_n2, W_in2, W_out2, b_n2, W_g2, W_gn2, W_ge2, b_g2)
    return (n2, e2, g2.reshape(H2))


# speedup vs baseline: 1.2745x; 1.2745x over previous
import functools

import jax
import jax.numpy as jnp
from jax import lax
from jax.experimental import pallas as pl
from jax.experimental.pallas import tpu as pltpu
from jax.experimental.pallas import tpu_sc as plsc

N = 10000
E = 320000
DN = 128
DE = 16
DG = 128
H1 = 256
H2 = 128

TE = 4000
GE = E // TE
TN = 2000
GN = N // TN
CHUNK = 128
NCHUNKS = E // CHUNK
NSUB = 16
CPS_BASE = NCHUNKS // NSUB
CPS_REM = NCHUNKS % NSUB

_DEBUG_JNP_SEG = False
_DEBUG_JNP_EDGE = False


def _edge_body(edges_ref, we1_ref, be1_ref, we2_ref, be2_ref,
               e1ab_ref, e2_ref, s1_ref, s2_ref):
    i = pl.program_id(0)
    e1 = jnp.maximum(
        jnp.dot(edges_ref[...], we1_ref[...],
                preferred_element_type=jnp.float32) + be1_ref[...], 0.0)
    e1ab_ref[0] = e1[:, :H1 // 2]
    e1ab_ref[1] = e1[:, H1 // 2:]
    e2 = jnp.maximum(
        jnp.dot(e1, we2_ref[...],
                preferred_element_type=jnp.float32) + be2_ref[...], 0.0)
    e2_ref[...] = e2

    @pl.when(i == 0)
    def _():
        s1_ref[...] = jnp.zeros_like(s1_ref)
        s2_ref[...] = jnp.zeros_like(s2_ref)

    s1_ref[...] += jnp.sum(e1, axis=0, keepdims=True)
    s2_ref[...] += jnp.sum(e2, axis=0, keepdims=True)


def _edge_stage(edges, W_e1, b_e1, W_e2, b_e2):
    return pl.pallas_call(
        _edge_body,
        grid=(GE,),
        in_specs=[
            pl.BlockSpec((TE, DE), lambda i: (i, 0)),
            pl.BlockSpec((DE, H1), lambda i: (0, 0)),
            pl.BlockSpec((1, H1), lambda i: (0, 0)),
            pl.BlockSpec((H1, H2), lambda i: (0, 0)),
            pl.BlockSpec((1, H2), lambda i: (0, 0)),
        ],
        out_specs=[
            pl.BlockSpec((2, TE, H1 // 2), lambda i: (0, i, 0)),
            pl.BlockSpec((TE, H2), lambda i: (i, 0)),
            pl.BlockSpec((1, H1), lambda i: (0, 0)),
            pl.BlockSpec((1, H2), lambda i: (0, 0)),
        ],
        out_shape=[
            jax.ShapeDtypeStruct((2, E, H1 // 2), jnp.float32),
            jax.ShapeDtypeStruct((E, H2), jnp.float32),
            jax.ShapeDtypeStruct((1, H1), jnp.float32),
            jax.ShapeDtypeStruct((1, H2), jnp.float32),
        ],
        compiler_params=pltpu.CompilerParams(
            dimension_semantics=("arbitrary",)),
    )(edges, W_e1, b_e1.reshape(1, H1), W_e2, b_e2.reshape(1, H2))


def _zero_fill(ref, rows, width):
    z = jnp.zeros((16,), jnp.float32)

    def body(i, _):
        for k in range(width // 16):
            ref[i, pl.ds(k * 16, 16)] = z
        return 0

    lax.fori_loop(0, rows, body, 0, unroll=False)


def _zero_shared(acc, zbuf, rows_total, s):
    zrows = zbuf.shape[0]
    stripe = rows_total // NSUB
    reps = stripe // zrows
    base = s * stripe
    for r in range(reps):
        pltpu.sync_copy(zbuf, acc.at[pl.ds(base + r * zrows, zrows), :])


NBUF = 2
STEPS = (CPS_BASE + (1 if CPS_REM else 0) + NBUF - 1) // NBUF


def _pipelined_scatter(idx_hbm, data_slice, idx_bufs, data_bufs,
                       isems, dsems, acc, s):
    start = s * CPS_BASE + jnp.minimum(s, CPS_REM)
    cnt = CPS_BASE + jnp.where(s < CPS_REM, 1, 0)
    end = start + cnt

    def issue(b, k):
        pltpu.async_copy(idx_hbm.at[pl.ds(k * CHUNK, CHUNK)],
                         idx_bufs[b], isems[b])
        if data_bufs is not None:
            pltpu.async_copy(data_slice(k), data_bufs[b], dsems[b])

    def wait(b):
        pltpu.make_async_copy(idx_hbm.at[pl.ds(0, CHUNK)],
                              idx_bufs[b], isems[b]).wait()
        if data_bufs is not None:
            pltpu.make_async_copy(data_slice(start), data_bufs[b],
                                  dsems[b]).wait()

    def scatter(b):
        src = data_bufs[b] if data_bufs is not None else data_slice
        pltpu.sync_copy(src, acc.at[idx_bufs[b]], add=True)

    for b in range(NBUF):
        @pl.when(start + b < end)
        def _(b=b):
            issue(b, start + b)

    def step_body(q, _):
        base = start + q * NBUF
        for b in range(NBUF):
            k = base + b

            @pl.when(k < end)
            def _(b=b, k=k):
                wait(b)
                scatter(b)

            @pl.when(k + NBUF < end)
            def _(b=b, k=k):
                issue(b, k + NBUF)

        return 0

    lax.fori_loop(0, STEPS, step_body, 0, unroll=False)


def _seg_sum_e1(data, idx2d, which):
    mesh = plsc.VectorSubcoreMesh(core_axis_name="c", subcore_axis_name="s")
    D2 = H1 // 2

    @functools.partial(
        pl.kernel,
        mesh=mesh,
        out_type=jax.ShapeDtypeStruct((2, N, D2), jnp.float32),
        scratch_types=[
            [pltpu.VMEM((CHUNK, D2), jnp.float32)] * NBUF,
            [pltpu.VMEM((CHUNK,), jnp.int32)] * NBUF,
            pltpu.VMEM((125, D2), jnp.float32),
            pltpu.VMEM_SHARED((N, D2), jnp.float32),
            [pltpu.SemaphoreType.DMA] * NBUF,
            [pltpu.SemaphoreType.DMA] * NBUF,
        ],
        name=f"seg_sum_e1_{which}",
    )
    def k(data_hbm, idx_hbm, out_hbm, data_vs, idx_vs, zbuf, acc, iss, dss):
        c = lax.axis_index("c")
        s = lax.axis_index("s")
        _zero_fill(zbuf, 125, D2)
        _zero_shared(acc, zbuf, N, s)
        plsc.subcore_barrier()

        def data_slice(k):
            return data_hbm.at[c, pl.ds(k * CHUNK, CHUNK), :]

        _pipelined_scatter(idx_hbm, data_slice, idx_vs, data_vs,
                           iss, dss, acc, s)
        plsc.subcore_barrier()

        @pl.when(s == 0)
        def _():
            pltpu.sync_copy(acc, out_hbm.at[c])

    return k(data, idx2d)


def _seg_sum_e2(data, ridx2d, sidx2d):
    mesh = plsc.VectorSubcoreMesh(core_axis_name="c", subcore_axis_name="s")

    @functools.partial(
        pl.kernel,
        mesh=mesh,
        out_type=jax.ShapeDtypeStruct((2, N, H2), jnp.float32),
        scratch_types=[
            [pltpu.VMEM((CHUNK, H2), jnp.float32)] * NBUF,
            [pltpu.VMEM((CHUNK,), jnp.int32)] * NBUF,
            pltpu.VMEM((125, H2), jnp.float32),
            pltpu.VMEM_SHARED((N, H2), jnp.float32),
            [pltpu.SemaphoreType.DMA] * NBUF,
            [pltpu.SemaphoreType.DMA] * NBUF,
        ],
        name="seg_sum_e2",
    )
    def k(data_hbm, ridx_hbm, sidx_hbm, out_hbm, data_vs, idx_vs,
          zbuf, acc, iss, dss):
        c = lax.axis_index("c")
        s = lax.axis_index("s")
        _zero_fill(zbuf, 125, H2)
        _zero_shared(acc, zbuf, N, s)
        plsc.subcore_barrier()

        def data_slice(k):
            return data_hbm.at[pl.ds(k * CHUNK, CHUNK), :]

        @pl.when(c == 0)
        def _():
            _pipelined_scatter(ridx_hbm, data_slice, idx_vs, data_vs,
                               iss, dss, acc, s)

        @pl.when(c == 1)
        def _():
            _pipelined_scatter(sidx_hbm, data_slice, idx_vs, data_vs,
                               iss, dss, acc, s)

        plsc.subcore_barrier()

        @pl.when(s == 0)
        def _():
            pltpu.sync_copy(acc, out_hbm.at[c])

    return k(data, ridx2d, sidx2d)


def _counts(ridx2d, sidx2d):
    mesh = plsc.VectorSubcoreMesh(core_axis_name="c", subcore_axis_name="s")

    @functools.partial(
        pl.kernel,
        mesh=mesh,
        out_type=jax.ShapeDtypeStruct((2, N, H2), jnp.float32),
        scratch_types=[
            pltpu.VMEM((CHUNK, H2), jnp.float32),
            [pltpu.VMEM((CHUNK,), jnp.int32)] * NBUF,
            [pltpu.SemaphoreType.DMA] * NBUF,
            pltpu.VMEM((125, H2), jnp.float32),
            pltpu.VMEM_SHARED((N, H2), jnp.float32),
        ],
        name="idx_counts",
    )
    def k(ridx_hbm, sidx_hbm, out_hbm, ones_v, idx_vs, iss, zbuf, acc):
        c = lax.axis_index("c")
        s = lax.axis_index("s")
        _zero_fill(zbuf, 125, H2)
        one = jnp.ones((16,), jnp.float32)

        def ones_body(i, _):
            for kk in range(H2 // 16):
                ones_v[i, pl.ds(kk * 16, 16)] = one
            return 0

        lax.fori_loop(0, CHUNK, ones_body, 0, unroll=False)
        _zero_shared(acc, zbuf, N, s)
        plsc.subcore_barrier()

        @pl.when(c == 0)
        def _():
            _pipelined_scatter(ridx_hbm, ones_v, idx_vs, None,
                               iss, None, acc, s)

        @pl.when(c == 1)
        def _():
            _pipelined_scatter(sidx_hbm, ones_v, idx_vs, None,
                               iss, None, acc, s)

        plsc.subcore_barrier()

        @pl.when(s == 0)
        def _():
            pltpu.sync_copy(acc, out_hbm.at[c])

    return k(ridx2d, sidx2d)


def _node_body(nodes_ref, g_ref, i1_ref, o1_ref, io2_ref,
               cnt_ref, s1_ref, s2_ref,
               wn1_ref, win1_ref, wout1_ref, bn1_ref,
               wg1_ref, wgn1_ref, wge1_ref, bg1_ref,
               wn2_ref, win2_ref, wout2_ref, bn2_ref,
               wg2_ref, wgn2_ref, wge2_ref, bg2_ref,
               n2_ref, g2_ref, sn1_ref, sn2_ref):
    i = pl.program_id(0)
    rr = 1.0 / jnp.maximum(cnt_ref[0, :, 0:1], 1.0)
    rs = 1.0 / jnp.maximum(cnt_ref[1, :, 0:1], 1.0)

    def dot(a, b):
        return jnp.dot(a, b, preferred_element_type=jnp.float32)

    h = H1 // 2
    n1 = dot(nodes_ref[...], wn1_ref[...])
    n1 += dot(i1_ref[0] * rr, win1_ref[0:h]) + dot(i1_ref[1] * rr, win1_ref[h:])
    n1 += dot(o1_ref[0] * rs, wout1_ref[0:h]) + dot(o1_ref[1] * rs, wout1_ref[h:])
    n1 = jnp.maximum(n1 + bn1_ref[...], 0.0)

    n2 = dot(n1, wn2_ref[...])
    n2 += dot(io2_ref[0] * rr, win2_ref[...])
    n2 += dot(io2_ref[1] * rs, wout2_ref[...])
    n2 = jnp.maximum(n2 + bn2_ref[...], 0.0)
    n2_ref[...] = n2

    @pl.when(i == 0)
    def _():
        sn1_ref[...] = jnp.zeros_like(sn1_ref)
        sn2_ref[...] = jnp.zeros_like(sn2_ref)

    sn1_ref[...] += jnp.sum(n1, axis=0, keepdims=True)
    sn2_ref[...] += jnp.sum(n2, axis=0, keepdims=True)

    @pl.when(i == GN - 1)
    def _():
        g1 = dot(g_ref[...], wg1_ref[...])
        g1 += dot(sn1_ref[...] * (1.0 / N), wgn1_ref[...])
        g1 += dot(s1_ref[...] * (1.0 / E), wge1_ref[...])
        g1 = jnp.maximum(g1 + bg1_ref[...], 0.0)
        g2 = dot(g1, wg2_ref[...])
        g2 += dot(sn2_ref[...] * (1.0 / N), wgn2_ref[...])
        g2 += dot(s2_ref[...] * (1.0 / E), wge2_ref[...])
        g2_ref[...] = jnp.maximum(g2 + bg2_ref[...], 0.0)


def _node_stage(nodes, globals_, i1, o1, io2, cnt, s1, s2,
                W_n1, W_in1, W_out1, b_n1, W_g1, W_gn1, W_ge1, b_g1,
                W_n2, W_in2, W_out2, b_n2, W_g2, W_gn2, W_ge2, b_g2):
    full = lambda a, b: pl.BlockSpec((a, b), lambda i: (0, 0))
    row = lambda w: pl.BlockSpec((TN, w), lambda i: (i, 0))
    split = lambda w: pl.BlockSpec((2, TN, w), lambda i: (0, i, 0))
    return pl.pallas_call(
        _node_body,
        grid=(GN,),
        in_specs=[
            row(DN), full(1, DG),
            split(H1 // 2), split(H1 // 2), split(H2),
            pl.BlockSpec((2, TN, H2), lambda i: (0, i, 0)),
            full(1, H1), full(1, H2),
            full(DN, H1), full(H1, H1), full(H1, H1), full(1, H1),
            full(DG, H1), full(H1, H1), full(H1, H1), full(1, H1),
            full(H1, H2), full(H2, H2), full(H2, H2), full(1, H2),
            full(H1, H2), full(H2, H2), full(H2, H2), full(1, H2),
        ],
        out_specs=[
            pl.BlockSpec((TN, H2), lambda i: (i, 0)),
            pl.BlockSpec((1, H2), lambda i: (0, 0)),
        ],
        out_shape=[
            jax.ShapeDtypeStruct((N, H2), jnp.float32),
            jax.ShapeDtypeStruct((1, H2), jnp.float32),
        ],
        scratch_shapes=[
            pltpu.VMEM((1, H1), jnp.float32),
            pltpu.VMEM((1, H2), jnp.float32),
        ],
        compiler_params=pltpu.CompilerParams(
            dimension_semantics=("arbitrary",)),
    )(nodes, globals_.reshape(1, DG), i1, o1, io2, cnt, s1, s2,
      W_n1, W_in1, W_out1, b_n1.reshape(1, H1),
      W_g1, W_gn1, W_ge1, b_g1.reshape(1, H1),
      W_n2, W_in2, W_out2, b_n2.reshape(1, H2),
      W_g2, W_gn2, W_ge2, b_g2.reshape(1, H2))


def kernel(nodes, edges, globals_, senders, receivers,
           W_e1, b_e1, W_n1, W_in1, W_out1, b_n1, W_g1, W_gn1, W_ge1, b_g1,
           W_e2, b_e2, W_n2, W_in2, W_out2, b_n2, W_g2, W_gn2, W_ge2, b_g2):
    ridx2d = receivers.astype(jnp.int32)
    sidx2d = senders.astype(jnp.int32)

    cnt_sc = _counts(ridx2d, sidx2d)
    if _DEBUG_JNP_EDGE:
        e1 = jax.nn.relu(edges @ W_e1 + b_e1)
        e2 = jax.nn.relu(e1 @ W_e2 + b_e2)
        e1ab = jnp.stack([e1[:, :H1 // 2], e1[:, H1 // 2:]], axis=0)
        s1 = jnp.sum(e1, axis=0, keepdims=True)
        s2 = jnp.sum(e2, axis=0, keepdims=True)
    else:
        e1ab, e2, s1, s2 = _edge_stage(edges, W_e1, b_e1, W_e2, b_e2)
    if _DEBUG_JNP_SEG:
        recv = receivers.astype(jnp.int32)
        send = senders.astype(jnp.int32)
        e1 = jnp.concatenate([e1ab[0], e1ab[1]], axis=1)
        seg = lambda d, x: jax.ops.segment_sum(d, x, num_segments=N)
        sp = lambda x: jnp.stack([x[:, :H1 // 2], x[:, H1 // 2:]], axis=0)
        i1 = sp(seg(e1, recv))
        o1 = sp(seg(e1, send))
        io2 = jnp.stack([seg(e2, recv), seg(e2, send)], axis=0)
        ones = jnp.ones((E,), jnp.float32)
        cnt = jnp.stack([jnp.tile(seg(ones, recv)[:, None], (1, H2)),
                         jnp.tile(seg(ones, send)[:, None], (1, H2))], axis=0)
    else:
        i1 = _seg_sum_e1(e1ab, ridx2d, "recv")
        o1 = _seg_sum_e1(e1ab, sidx2d, "send")
        io2 = _seg_sum_e2(e2, ridx2d, sidx2d)
        cnt = cnt_sc

    n2, g2 = _node_stage(
        nodes, globals_, i1, o1, io2, cnt, s1, s2,
        W_n1, W_in1, W_out1, b_n1, W_g1, W_gn1, W_ge1, b_g1,
        W_n2, W_in2, W_out2, b_n2, W_g2, W_gn2, W_ge2, b_g2)
    return (n2, e2, g2.reshape(H2))


# speedup vs baseline: 1.2802x; 1.0045x over previous
import functools

import jax
import jax.numpy as jnp
from jax import lax
from jax.experimental import pallas as pl
from jax.experimental.pallas import tpu as pltpu
from jax.experimental.pallas import tpu_sc as plsc

N = 10000
E = 320000
DN = 128
DE = 16
DG = 128
H1 = 256
H2 = 128

TE = 4000
GE = E // TE
TN = 2000
GN = N // TN
CHUNK = 128
NCHUNKS = E // CHUNK
NSUB = 16
CPS_BASE = NCHUNKS // NSUB
CPS_REM = NCHUNKS % NSUB


def _edge_body(edges_ref, we1_ref, be1_ref, we2_ref, be2_ref,
               e1ab_ref, e2_ref, s1_ref, s2_ref):
    i = pl.program_id(0)
    e1 = jnp.maximum(
        jnp.dot(edges_ref[...], we1_ref[...],
                preferred_element_type=jnp.float32) + be1_ref[...], 0.0)
    e1ab_ref[0] = e1[:, :H1 // 2]
    e1ab_ref[1] = e1[:, H1 // 2:]
    e2 = jnp.maximum(
        jnp.dot(e1, we2_ref[...],
                preferred_element_type=jnp.float32) + be2_ref[...], 0.0)
    e2_ref[...] = e2

    @pl.when(i == 0)
    def _():
        s1_ref[...] = jnp.zeros_like(s1_ref)
        s2_ref[...] = jnp.zeros_like(s2_ref)

    s1_ref[...] += jnp.sum(e1, axis=0, keepdims=True)
    s2_ref[...] += jnp.sum(e2, axis=0, keepdims=True)


def _edge_stage(edges, W_e1, b_e1, W_e2, b_e2):
    return pl.pallas_call(
        _edge_body,
        grid=(GE,),
        in_specs=[
            pl.BlockSpec((TE, DE), lambda i: (i, 0)),
            pl.BlockSpec((DE, H1), lambda i: (0, 0)),
            pl.BlockSpec((1, H1), lambda i: (0, 0)),
            pl.BlockSpec((H1, H2), lambda i: (0, 0)),
            pl.BlockSpec((1, H2), lambda i: (0, 0)),
        ],
        out_specs=[
            pl.BlockSpec((2, TE, H1 // 2), lambda i: (0, i, 0)),
            pl.BlockSpec((TE, H2), lambda i: (i, 0)),
            pl.BlockSpec((1, H1), lambda i: (0, 0)),
            pl.BlockSpec((1, H2), lambda i: (0, 0)),
        ],
        out_shape=[
            jax.ShapeDtypeStruct((2, E, H1 // 2), jnp.float32),
            jax.ShapeDtypeStruct((E, H2), jnp.float32),
            jax.ShapeDtypeStruct((1, H1), jnp.float32),
            jax.ShapeDtypeStruct((1, H2), jnp.float32),
        ],
        compiler_params=pltpu.CompilerParams(
            dimension_semantics=("arbitrary",)),
    )(edges, W_e1, b_e1.reshape(1, H1), W_e2, b_e2.reshape(1, H2))


def _zero_fill(ref, rows, width):
    z = jnp.zeros((16,), jnp.float32)

    def body(i, _):
        for k in range(width // 16):
            ref[i, pl.ds(k * 16, 16)] = z
        return 0

    lax.fori_loop(0, rows, body, 0, unroll=False)


def _zero_shared(acc, zbuf, rows_total, s):
    zrows = zbuf.shape[0]
    stripe = rows_total // NSUB
    reps = stripe // zrows
    base = s * stripe
    for r in range(reps):
        pltpu.sync_copy(zbuf, acc.at[pl.ds(base + r * zrows, zrows), :])


NBUF = 2
STEPS = (CPS_BASE + (1 if CPS_REM else 0) + NBUF - 1) // NBUF


def _pipelined_scatter(idx_hbm, data_slice, idx_bufs, data_bufs,
                       isems, dsems, acc, s):
    start = s * CPS_BASE + jnp.minimum(s, CPS_REM)
    cnt = CPS_BASE + jnp.where(s < CPS_REM, 1, 0)
    end = start + cnt

    def issue(b, k):
        pltpu.async_copy(idx_hbm.at[pl.ds(k * CHUNK, CHUNK)],
                         idx_bufs[b], isems[b])
        if data_bufs is not None:
            pltpu.async_copy(data_slice(k), data_bufs[b], dsems[b])

    def wait(b):
        pltpu.make_async_copy(idx_hbm.at[pl.ds(0, CHUNK)],
                              idx_bufs[b], isems[b]).wait()
        if data_bufs is not None:
            pltpu.make_async_copy(data_slice(start), data_bufs[b],
                                  dsems[b]).wait()

    def scatter(b):
        src = data_bufs[b] if data_bufs is not None else data_slice
        pltpu.sync_copy(src, acc.at[idx_bufs[b]], add=True)

    for b in range(NBUF):
        @pl.when(start + b < end)
        def _(b=b):
            issue(b, start + b)

    def step_body(q, _):
        base = start + q * NBUF
        for b in range(NBUF):
            k = base + b

            @pl.when(k < end)
            def _(b=b, k=k):
                wait(b)
                scatter(b)

            @pl.when(k + NBUF < end)
            def _(b=b, k=k):
                issue(b, k + NBUF)

        return 0

    lax.fori_loop(0, STEPS, step_body, 0, unroll=False)


def _seg_sum_e1(data, idx1d, which):
    mesh = plsc.VectorSubcoreMesh(core_axis_name="c", subcore_axis_name="s")
    D2 = H1 // 2

    @functools.partial(
        pl.kernel,
        mesh=mesh,
        out_type=jax.ShapeDtypeStruct((2, N, D2), jnp.float32),
        scratch_types=[
            [pltpu.VMEM((CHUNK, D2), jnp.float32)] * NBUF,
            [pltpu.VMEM((CHUNK,), jnp.int32)] * NBUF,
            pltpu.VMEM((125, D2), jnp.float32),
            pltpu.VMEM_SHARED((N, D2), jnp.float32),
            [pltpu.SemaphoreType.DMA] * NBUF,
            [pltpu.SemaphoreType.DMA] * NBUF,
        ],
        name=f"seg_sum_e1_{which}",
    )
    def k(data_hbm, idx_hbm, out_hbm, data_vs, idx_vs, zbuf, acc, iss, dss):
        c = lax.axis_index("c")
        s = lax.axis_index("s")
        _zero_fill(zbuf, 125, D2)
        _zero_shared(acc, zbuf, N, s)
        plsc.subcore_barrier()

        def data_slice(k):
            return data_hbm.at[c, pl.ds(k * CHUNK, CHUNK), :]

        _pipelined_scatter(idx_hbm, data_slice, idx_vs, data_vs,
                           iss, dss, acc, s)
        plsc.subcore_barrier()

        @pl.when(s == 0)
        def _():
            pltpu.sync_copy(acc, out_hbm.at[c])

    return k(data, idx1d)


def _seg_sum_e2(data, ridx1d, sidx1d):
    mesh = plsc.VectorSubcoreMesh(core_axis_name="c", subcore_axis_name="s")

    @functools.partial(
        pl.kernel,
        mesh=mesh,
        out_type=jax.ShapeDtypeStruct((2, N, H2), jnp.float32),
        scratch_types=[
            [pltpu.VMEM((CHUNK, H2), jnp.float32)] * NBUF,
            [pltpu.VMEM((CHUNK,), jnp.int32)] * NBUF,
            pltpu.VMEM((125, H2), jnp.float32),
            pltpu.VMEM_SHARED((N, H2), jnp.float32),
            [pltpu.SemaphoreType.DMA] * NBUF,
            [pltpu.SemaphoreType.DMA] * NBUF,
        ],
        name="seg_sum_e2",
    )
    def k(data_hbm, ridx_hbm, sidx_hbm, out_hbm, data_vs, idx_vs,
          zbuf, acc, iss, dss):
        c = lax.axis_index("c")
        s = lax.axis_index("s")
        _zero_fill(zbuf, 125, H2)
        _zero_shared(acc, zbuf, N, s)
        plsc.subcore_barrier()

        def data_slice(k):
            return data_hbm.at[pl.ds(k * CHUNK, CHUNK), :]

        @pl.when(c == 0)
        def _():
            _pipelined_scatter(ridx_hbm, data_slice, idx_vs, data_vs,
                               iss, dss, acc, s)

        @pl.when(c == 1)
        def _():
            _pipelined_scatter(sidx_hbm, data_slice, idx_vs, data_vs,
                               iss, dss, acc, s)

        plsc.subcore_barrier()

        @pl.when(s == 0)
        def _():
            pltpu.sync_copy(acc, out_hbm.at[c])

    return k(data, ridx1d, sidx1d)


def _counts(ridx1d, sidx1d):
    mesh = plsc.VectorSubcoreMesh(core_axis_name="c", subcore_axis_name="s")

    @functools.partial(
        pl.kernel,
        mesh=mesh,
        out_type=jax.ShapeDtypeStruct((2, N, H2), jnp.float32),
        scratch_types=[
            pltpu.VMEM((CHUNK, H2), jnp.float32),
            [pltpu.VMEM((CHUNK,), jnp.int32)] * NBUF,
            [pltpu.SemaphoreType.DMA] * NBUF,
            pltpu.VMEM((125, H2), jnp.float32),
            pltpu.VMEM_SHARED((N, H2), jnp.float32),
        ],
        name="idx_counts",
    )
    def k(ridx_hbm, sidx_hbm, out_hbm, ones_v, idx_vs, iss, zbuf, acc):
        c = lax.axis_index("c")
        s = lax.axis_index("s")
        _zero_fill(zbuf, 125, H2)
        one = jnp.ones((16,), jnp.float32)

        def ones_body(i, _):
            for kk in range(H2 // 16):
                ones_v[i, pl.ds(kk * 16, 16)] = one
            return 0

        lax.fori_loop(0, CHUNK, ones_body, 0, unroll=False)
        _zero_shared(acc, zbuf, N, s)
        plsc.subcore_barrier()

        @pl.when(c == 0)
        def _():
            _pipelined_scatter(ridx_hbm, ones_v, idx_vs, None,
                               iss, None, acc, s)

        @pl.when(c == 1)
        def _():
            _pipelined_scatter(sidx_hbm, ones_v, idx_vs, None,
                               iss, None, acc, s)

        plsc.subcore_barrier()

        @pl.when(s == 0)
        def _():
            pltpu.sync_copy(acc, out_hbm.at[c])

    return k(ridx1d, sidx1d)


def _node_body(nodes_ref, g_ref, i1_ref, o1_ref, io2_ref,
               cnt_ref, s1_ref, s2_ref,
               wn1_ref, win1_ref, wout1_ref, bn1_ref,
               wg1_ref, wgn1_ref, wge1_ref, bg1_ref,
               wn2_ref, win2_ref, wout2_ref, bn2_ref,
               wg2_ref, wgn2_ref, wge2_ref, bg2_ref,
               n2_ref, g2_ref, sn1_ref, sn2_ref):
    i = pl.program_id(0)
    rr = 1.0 / jnp.maximum(cnt_ref[0, :, 0:1], 1.0)
    rs = 1.0 / jnp.maximum(cnt_ref[1, :, 0:1], 1.0)

    def dot(a, b):
        return jnp.dot(a, b, preferred_element_type=jnp.float32)

    h = H1 // 2
    n1 = dot(nodes_ref[...], wn1_ref[...])
    n1 += dot(i1_ref[0] * rr, win1_ref[0:h]) + dot(i1_ref[1] * rr, win1_ref[h:])
    n1 += dot(o1_ref[0] * rs, wout1_ref[0:h]) + dot(o1_ref[1] * rs, wout1_ref[h:])
    n1 = jnp.maximum(n1 + bn1_ref[...], 0.0)

    n2 = dot(n1, wn2_ref[...])
    n2 += dot(io2_ref[0] * rr, win2_ref[...])
    n2 += dot(io2_ref[1] * rs, wout2_ref[...])
    n2 = jnp.maximum(n2 + bn2_ref[...], 0.0)
    n2_ref[...] = n2

    @pl.when(i == 0)
    def _():
        sn1_ref[...] = jnp.zeros_like(sn1_ref)
        sn2_ref[...] = jnp.zeros_like(sn2_ref)

    sn1_ref[...] += jnp.sum(n1, axis=0, keepdims=True)
    sn2_ref[...] += jnp.sum(n2, axis=0, keepdims=True)

    @pl.when(i == GN - 1)
    def _():
        g1 = dot(g_ref[...], wg1_ref[...])
        g1 += dot(sn1_ref[...] * (1.0 / N), wgn1_ref[...])
        g1 += dot(s1_ref[...] * (1.0 / E), wge1_ref[...])
        g1 = jnp.maximum(g1 + bg1_ref[...], 0.0)
        g2 = dot(g1, wg2_ref[...])
        g2 += dot(sn2_ref[...] * (1.0 / N), wgn2_ref[...])
        g2 += dot(s2_ref[...] * (1.0 / E), wge2_ref[...])
        g2_ref[...] = jnp.maximum(g2 + bg2_ref[...], 0.0)


def _node_stage(nodes, globals_, i1, o1, io2, cnt, s1, s2,
                W_n1, W_in1, W_out1, b_n1, W_g1, W_gn1, W_ge1, b_g1,
                W_n2, W_in2, W_out2, b_n2, W_g2, W_gn2, W_ge2, b_g2):
    full = lambda a, b: pl.BlockSpec((a, b), lambda i: (0, 0))
    row = lambda w: pl.BlockSpec((TN, w), lambda i: (i, 0))
    split = lambda w: pl.BlockSpec((2, TN, w), lambda i: (0, i, 0))
    return pl.pallas_call(
        _node_body,
        grid=(GN,),
        in_specs=[
            row(DN), full(1, DG),
            split(H1 // 2), split(H1 // 2), split(H2),
            pl.BlockSpec((2, TN, H2), lambda i: (0, i, 0)),
            full(1, H1), full(1, H2),
            full(DN, H1), full(H1, H1), full(H1, H1), full(1, H1),
            full(DG, H1), full(H1, H1), full(H1, H1), full(1, H1),
            full(H1, H2), full(H2, H2), full(H2, H2), full(1, H2),
            full(H1, H2), full(H2, H2), full(H2, H2), full(1, H2),
        ],
        out_specs=[
            pl.BlockSpec((TN, H2), lambda i: (i, 0)),
            pl.BlockSpec((1, H2), lambda i: (0, 0)),
        ],
        out_shape=[
            jax.ShapeDtypeStruct((N, H2), jnp.float32),
            jax.ShapeDtypeStruct((1, H2), jnp.float32),
        ],
        scratch_shapes=[
            pltpu.VMEM((1, H1), jnp.float32),
            pltpu.VMEM((1, H2), jnp.float32),
        ],
        compiler_params=pltpu.CompilerParams(
            dimension_semantics=("arbitrary",)),
    )(nodes, globals_.reshape(1, DG), i1, o1, io2, cnt, s1, s2,
      W_n1, W_in1, W_out1, b_n1.reshape(1, H1),
      W_g1, W_gn1, W_ge1, b_g1.reshape(1, H1),
      W_n2, W_in2, W_out2, b_n2.reshape(1, H2),
      W_g2, W_gn2, W_ge2, b_g2.reshape(1, H2))


def kernel(nodes, edges, globals_, senders, receivers,
           W_e1, b_e1, W_n1, W_in1, W_out1, b_n1, W_g1, W_gn1, W_ge1, b_g1,
           W_e2, b_e2, W_n2, W_in2, W_out2, b_n2, W_g2, W_gn2, W_ge2, b_g2):
    ridx = receivers.astype(jnp.int32)
    sidx = senders.astype(jnp.int32)

    cnt = _counts(ridx, sidx)
    e1ab, e2, s1, s2 = _edge_stage(edges, W_e1, b_e1, W_e2, b_e2)
    i1 = _seg_sum_e1(e1ab, ridx, "recv")
    o1 = _seg_sum_e1(e1ab, sidx, "send")
    io2 = _seg_sum_e2(e2, ridx, sidx)

    n2, g2 = _node_stage(
        nodes, globals_, i1, o1, io2, cnt, s1, s2,
        W_n1, W_in1, W_out1, b_n1, W_g1, W_gn1, W_ge1, b_g1,
        W_n2, W_in2, W_out2, b_n2, W_g2, W_gn2, W_ge2, b_g2)
    return (n2, e2, g2.reshape(H2))
